# baseline (speedup 1.0000x reference)
import jax
import jax.numpy as jnp
from jax import lax
from jax.experimental import pallas as pl
from jax.experimental.pallas import tpu as pltpu
from jax.experimental.pallas import tpu_sc as plsc

_NUM_CORES = 2
_NUM_SUBCORES = 16
_LANES = 16
_NW = _NUM_CORES * _NUM_SUBCORES

_D = 128
_BATCH = 4
_SEQ = 4096
_N = _BATCH * _SEQ
_PER_W = _N // _NW
_CHUNK = 128
_NCHUNK = _PER_W // _CHUNK


def _embed_body(tok_tab, pos_tab, idx, out,
                idx_v, tok_rows, pos_rows, out_rows,
                sem_g0, sem_g1, sem_s0, sem_s1):
    c = lax.axis_index("c")
    s = lax.axis_index("s")
    wid = s * _NUM_CORES + c
    sem_g = (sem_g0, sem_g1)
    sem_s = (sem_s0, sem_s1)
    pltpu.sync_copy(idx.at[wid], idx_v)

    gathers = {}
    stores = {}

    def start_gathers(j):
        b = j % 2
        gathers[j] = (
            pltpu.async_copy(tok_tab.at[idx_v.at[j]], tok_rows.at[b],
                             sem_g[b]),
            pltpu.async_copy(pos_tab.at[idx_v.at[_NCHUNK + j]],
                             pos_rows.at[b], sem_g[b]),
        )

    start_gathers(0)
    for j in range(_NCHUNK):
        b = j % 2
        if j + 1 < _NCHUNK:
            start_gathers(j + 1)
        ct, cp = gathers.pop(j)
        ct.wait()
        cp.wait()
        if j >= 2:
            stores.pop(j - 2).wait()

        def add_row(i, _):
            for q in range(_D // _LANES):
                sl = pl.ds(q * _LANES, _LANES)
                out_rows[b, i, sl] = tok_rows[b, i, sl] + pos_rows[b, i, sl]
            return 0

        lax.fori_loop(0, _CHUNK, add_row, 0, unroll=2)
        stores[j] = pltpu.async_copy(
            out_rows.at[b],
            out.at[pl.ds(wid * _PER_W + j * _CHUNK, _CHUNK)],
            sem_s[b])
    for j in sorted(stores):
        stores.pop(j).wait()


def _embed(tok_table, pos_table, idx):
    mesh = plsc.VectorSubcoreMesh(core_axis_name="c", subcore_axis_name="s")
    return pl.kernel(
        _embed_body,
        out_type=jax.ShapeDtypeStruct((_N, _D), jnp.float32),
        mesh=mesh,
        scratch_types=[
            pltpu.VMEM((2 * _NCHUNK, _CHUNK), jnp.int32),
            pltpu.VMEM((2, _CHUNK, _D), jnp.float32),
            pltpu.VMEM((2, _CHUNK, _D), jnp.float32),
            pltpu.VMEM((2, _CHUNK, _D), jnp.float32),
            pltpu.SemaphoreType.DMA,
            pltpu.SemaphoreType.DMA,
            pltpu.SemaphoreType.DMA,
            pltpu.SemaphoreType.DMA,
        ],
    )(tok_table, pos_table, idx)


def kernel(token_ids, position_ids, tok_table, pos_table):
    tid = token_ids.reshape(_NW, _NCHUNK, _CHUNK)
    pid = position_ids.reshape(_NW, _NCHUNK, _CHUNK)
    idx = jnp.concatenate([tid, pid], axis=1)
    out = _embed(tok_table, pos_table, idx)
    return out.reshape(_BATCH, _SEQ, _D)


# speedup vs baseline: 1.2229x; 1.2229x over previous
import jax
import jax.numpy as jnp
from jax import lax
from jax.experimental import pallas as pl
from jax.experimental.pallas import tpu as pltpu
from jax.experimental.pallas import tpu_sc as plsc

_NUM_CORES = 2
_NUM_SUBCORES = 16
_LANES = 16
_NW = _NUM_CORES * _NUM_SUBCORES

_D = 128
_BATCH = 4
_SEQ = 4096
_N = _BATCH * _SEQ
_PER_W = _N // _NW
_CHUNK = 128
_NCHUNK = _PER_W // _CHUNK


def _embed_body(tok_tab, pos_tab, idx, out,
                idx_v, rows, sem_g0, sem_g1, sem_s0, sem_s1):
    c = lax.axis_index("c")
    s = lax.axis_index("s")
    wid = s * _NUM_CORES + c
    sem_g = (sem_g0, sem_g1)
    sem_s = (sem_s0, sem_s1)
    pltpu.sync_copy(idx.at[wid], idx_v)

    stores = {}
    for j in range(_NCHUNK):
        b = j % 2
        if j >= 2:
            stores.pop(j - 2).wait()
        ct = pltpu.async_copy(tok_tab.at[idx_v.at[j]], rows.at[b], sem_g[b])
        ct.wait()
        cp = pltpu.async_copy(pos_tab.at[idx_v.at[_NCHUNK + j]], rows.at[b],
                              sem_g[b], add=True)
        cp.wait()
        stores[j] = pltpu.async_copy(
            rows.at[b],
            out.at[pl.ds(wid * _PER_W + j * _CHUNK, _CHUNK)],
            sem_s[b])
    for j in sorted(stores):
        stores.pop(j).wait()


def _embed(tok_table, pos_table, idx):
    mesh = plsc.VectorSubcoreMesh(core_axis_name="c", subcore_axis_name="s")
    return pl.kernel(
        _embed_body,
        out_type=jax.ShapeDtypeStruct((_N, _D), jnp.float32),
        mesh=mesh,
        scratch_types=[
            pltpu.VMEM((2 * _NCHUNK, _CHUNK), jnp.int32),
            pltpu.VMEM((2, _CHUNK, _D), jnp.float32),
            pltpu.SemaphoreType.DMA,
            pltpu.SemaphoreType.DMA,
            pltpu.SemaphoreType.DMA,
            pltpu.SemaphoreType.DMA,
        ],
    )(tok_table, pos_table, idx)


def kernel(token_ids, position_ids, tok_table, pos_table):
    tid = token_ids.reshape(_NW, _NCHUNK, _CHUNK)
    pid = position_ids.reshape(_NW, _NCHUNK, _CHUNK)
    idx = jnp.concatenate([tid, pid], axis=1)
    out = _embed(tok_table, pos_table, idx)
    return out.reshape(_BATCH, _SEQ, _D)


# speedup vs baseline: 1.3363x; 1.0928x over previous
import jax
import jax.numpy as jnp
from jax import lax
from jax.experimental import pallas as pl
from jax.experimental.pallas import tpu as pltpu
from jax.experimental.pallas import tpu_sc as plsc

_NUM_CORES = 2
_NUM_SUBCORES = 16
_NW = _NUM_CORES * _NUM_SUBCORES

_D = 128
_BATCH = 4
_SEQ = 4096
_N = _BATCH * _SEQ
_PER_W = _N // _NW
_CHUNK = 128
_NCHUNK = _PER_W // _CHUNK


def _embed_body(tok_tab, pos_tab, idx, out,
                idx_v, rows, sem_g0, sem_g1, sem_g2, sem_g3, sem_s):
    c = lax.axis_index("c")
    s = lax.axis_index("s")
    wid = s * _NUM_CORES + c
    sem_g = (sem_g0, sem_g1, sem_g2, sem_g3)
    pltpu.sync_copy(idx.at[wid], idx_v)

    toks = [pltpu.async_copy(tok_tab.at[idx_v.at[j]], rows.at[j], sem_g[j])
            for j in range(_NCHUNK)]
    adds = []
    for j in range(_NCHUNK):
        toks[j].wait()
        adds.append(pltpu.async_copy(pos_tab.at[idx_v.at[_NCHUNK + j]],
                                     rows.at[j], sem_g[j], add=True))
    stores = []
    for j in range(_NCHUNK):
        adds[j].wait()
        stores.append(pltpu.async_copy(
            rows.at[j],
            out.at[pl.ds(wid * _PER_W + j * _CHUNK, _CHUNK)],
            sem_s))
    for st in stores:
        st.wait()


def _embed(tok_table, pos_table, idx):
    mesh = plsc.VectorSubcoreMesh(core_axis_name="c", subcore_axis_name="s")
    return pl.kernel(
        _embed_body,
        out_type=jax.ShapeDtypeStruct((_N, _D), jnp.float32),
        mesh=mesh,
        scratch_types=[
            pltpu.VMEM((2 * _NCHUNK, _CHUNK), jnp.int32),
            pltpu.VMEM((_NCHUNK, _CHUNK, _D), jnp.float32),
            pltpu.SemaphoreType.DMA,
            pltpu.SemaphoreType.DMA,
            pltpu.SemaphoreType.DMA,
            pltpu.SemaphoreType.DMA,
            pltpu.SemaphoreType.DMA,
        ],
    )(tok_table, pos_table, idx)


def kernel(token_ids, position_ids, tok_table, pos_table):
    tid = token_ids.reshape(_NW, _NCHUNK, _CHUNK)
    pid = position_ids.reshape(_NW, _NCHUNK, _CHUNK)
    idx = jnp.concatenate([tid, pid], axis=1)
    out = _embed(tok_table, pos_table, idx)
    return out.reshape(_BATCH, _SEQ, _D)
